# baseline (speedup 1.0000x reference)
import functools

import numpy as np
import jax
import jax.numpy as jnp
from jax import lax
from jax.experimental import pallas as pl
from jax.experimental.pallas import tpu as pltpu
from jax.experimental.pallas import tpu_sc as plsc

SUBSAMPLE_SIZE = 1024
GROUP_LIMIT = 64
N_POINTS = 16384

_LANES = 16
_NW = 32
_S_PER = SUBSAMPLE_SIZE // _NW
_CAND = 512
_T04 = float(np.float32(0.040000003))


def _fps_body(x_ref, y_ref, z_ref, cx_ref, cy_ref, cz_ref):
    x = x_ref[...]
    y = y_ref[...]
    z = z_ref[...]
    ri = lax.broadcasted_iota(jnp.int32, (128, 128), 0)
    ci = lax.broadcasted_iota(jnp.int32, (128, 128), 1)
    flat = ri * 128 + ci

    def extract(a, idx):
        return jnp.sum(jnp.where(flat == idx, a, 0.0))

    lx0 = extract(x, 0)
    ly0 = extract(y, 0)
    lz0 = extract(z, 0)
    cx_ref[0] = lx0
    cy_ref[0] = ly0
    cz_ref[0] = lz0
    dmin0 = jnp.full((128, 128), jnp.inf, dtype=jnp.float32)

    def body(i, carry):
        dmin, lx, ly, lz = carry
        dx = x - lx
        dy = y - ly
        dz = z - lz
        d = dx * dx + dy * dy + dz * dz
        dmin = jnp.minimum(dmin, d)
        m = jnp.max(dmin)
        idx = jnp.min(jnp.where(dmin == m, flat, jnp.int32(1 << 30)))
        nlx = extract(x, idx)
        nly = extract(y, idx)
        nlz = extract(z, idx)
        cx_ref[i] = nlx
        cy_ref[i] = nly
        cz_ref[i] = nlz
        return dmin, nlx, nly, nlz

    lax.fori_loop(1, SUBSAMPLE_SIZE, body, (dmin0, lx0, ly0, lz0))


def _fps_pallas(vertices):
    xp = vertices[:, 0].reshape(128, 128)
    yp = vertices[:, 1].reshape(128, 128)
    zp = vertices[:, 2].reshape(128, 128)
    s = jax.ShapeDtypeStruct((SUBSAMPLE_SIZE,), jnp.float32)
    return pl.pallas_call(
        _fps_body,
        out_specs=[pl.BlockSpec(memory_space=pltpu.SMEM)] * 3,
        out_shape=[s, s, s],
    )(xp, yp, zp)


def _bq_body(xs_h, ys_h, zs_h, bxs_h, bys_h, bzs_h, sqv_h,
             bcx_h, bcy_h, bcz_h, sqc_h,
             gx_h, gy_h, gz_h, cnt_h,
             xs_v, ys_v, zs_v, bxs_v, bys_v, bzs_v, sqv_v,
             bcxv, bcyv, bczv, sqcv,
             cid_v, ca2_v, cgx_v, cgy_v, cgz_v,
             ogx_v, ogy_v, ogz_v, ocnt_v):
    i32 = jnp.int32
    f32 = jnp.float32
    wid = lax.axis_index("s") * 2 + lax.axis_index("c")
    row0 = wid * _S_PER
    pltpu.sync_copy(xs_h, xs_v)
    pltpu.sync_copy(ys_h, ys_v)
    pltpu.sync_copy(zs_h, zs_v)
    pltpu.sync_copy(bxs_h, bxs_v)
    pltpu.sync_copy(bys_h, bys_v)
    pltpu.sync_copy(bzs_h, bzs_v)
    pltpu.sync_copy(sqv_h, sqv_v)
    pltpu.sync_copy(bcx_h.at[pl.ds(row0 * _LANES, _S_PER * _LANES)], bcxv)
    pltpu.sync_copy(bcy_h.at[pl.ds(row0 * _LANES, _S_PER * _LANES)], bcyv)
    pltpu.sync_copy(bcz_h.at[pl.ds(row0 * _LANES, _S_PER * _LANES)], bczv)
    pltpu.sync_copy(sqc_h.at[pl.ds(row0 * _LANES, _S_PER * _LANES)], sqcv)

    lane = lax.broadcasted_iota(i32, (_LANES,), 0)
    zero16i = jnp.zeros((_LANES,), i32)
    one16i = jnp.ones((_LANES,), i32)
    t04 = jnp.full((_LANES,), _T04, f32)
    inf16 = jnp.full((_LANES,), jnp.inf, f32)

    def per_centroid(j, _):
        cxs = bcxv[pl.ds(j * _LANES, _LANES)]
        cys = bcyv[pl.ds(j * _LANES, _LANES)]
        czs = bczv[pl.ds(j * _LANES, _LANES)]
        sqcs = sqcv[pl.ds(j * _LANES, _LANES)]

        def scan_it(i, cnt_v):
            for u in range(4):
                base = (i * 4 + u) * _LANES
                xv = bxs_v[pl.ds(base, _LANES)]
                yv = bys_v[pl.ds(base, _LANES)]
                zv = bzs_v[pl.ds(base, _LANES)]
                sv = sqv_v[pl.ds(base, _LANES)]
                dot = xv * cxs + yv * cys + zv * czs
                a2 = jnp.abs(sqcs - 2.0 * dot + sv)
                m = a2 <= t04
                cum = plsc.cumsum(jnp.where(m, one16i, zero16i))
                slot = cnt_v + cum - 1
                m2 = jnp.logical_and(m, slot < _CAND)
                plsc.store_scatter(cid_v, [slot], lane + base, mask=m2)
                cnt_v = cnt_v + plsc.all_reduce_population_count(m2)
            return cnt_v

        cnt_v = lax.fori_loop(0, N_POINTS // _LANES // 4, scan_it, zero16i)
        c = jnp.max(cnt_v)
        cbase = j * GROUP_LIMIT
        zf = jnp.zeros((_LANES,), f32)
        for k in range(GROUP_LIMIT // _LANES):
            ogx_v[pl.ds(cbase + k * _LANES, _LANES)] = zf
            ogy_v[pl.ds(cbase + k * _LANES, _LANES)] = zf
            ogz_v[pl.ds(cbase + k * _LANES, _LANES)] = zf
        nch = (c + _LANES - 1) // _LANES

        def gath(i, _):
            valid = (i * _LANES + lane) < cnt_v
            idxv = cid_v[pl.ds(i * _LANES, _LANES)]
            idxc = jnp.where(valid, idxv, zero16i)
            bxv = plsc.load_gather(bxs_v, [idxc])
            byv = plsc.load_gather(bys_v, [idxc])
            bzv = plsc.load_gather(bzs_v, [idxc])
            sv = plsc.load_gather(sqv_v, [idxc])
            dot = bxv * cxs + byv * cys + bzv * czs
            a2 = jnp.abs(sqcs - 2.0 * dot + sv)
            ca2_v[pl.ds(i * _LANES, _LANES)] = jnp.where(valid, a2, inf16)
            cgx_v[pl.ds(i * _LANES, _LANES)] = plsc.load_gather(xs_v, [idxc])
            cgy_v[pl.ds(i * _LANES, _LANES)] = plsc.load_gather(ys_v, [idxc])
            cgz_v[pl.ds(i * _LANES, _LANES)] = plsc.load_gather(zs_v, [idxc])
            return 0

        lax.fori_loop(0, nch, gath, 0)

        def bsearch():
            def bs(_, lohi):
                lo, hi = lohi
                mid = 0.5 * (lo + hi)

                def cntit(i, acc):
                    a2v = ca2_v[pl.ds(i * _LANES, _LANES)]
                    return acc + plsc.all_reduce_population_count(a2v <= mid)

                cv = lax.fori_loop(0, nch, cntit, zero16i)
                ge = cv >= GROUP_LIMIT
                return jnp.where(ge, lo, mid), jnp.where(ge, mid, hi)

            lo0 = jnp.zeros((_LANES,), f32)
            return lax.fori_loop(0, 40, bs, (lo0, t04))[1]

        t = lax.cond(c > GROUP_LIMIT, bsearch, lambda: t04)

        def sel(i, taken):
            valid = (i * _LANES + lane) < cnt_v
            a2v = ca2_v[pl.ds(i * _LANES, _LANES)]
            m = jnp.logical_and(a2v <= t, valid)
            cum = plsc.cumsum(jnp.where(m, one16i, zero16i))
            slot = taken + cum - 1
            m2 = jnp.logical_and(m, slot < GROUP_LIMIT)
            gxv = cgx_v[pl.ds(i * _LANES, _LANES)]
            gyv = cgy_v[pl.ds(i * _LANES, _LANES)]
            gzv = cgz_v[pl.ds(i * _LANES, _LANES)]
            plsc.store_scatter(ogx_v, [cbase + slot], gxv, mask=m2)
            plsc.store_scatter(ogy_v, [cbase + slot], gyv, mask=m2)
            plsc.store_scatter(ogz_v, [cbase + slot], gzv, mask=m2)
            return taken + plsc.all_reduce_population_count(m2)

        lax.fori_loop(0, nch, sel, zero16i)
        ocnt_v[pl.ds(j * _LANES, _LANES)] = jnp.minimum(cnt_v, GROUP_LIMIT)
        return 0

    lax.fori_loop(0, _S_PER, per_centroid, 0)
    pltpu.sync_copy(ogx_v, gx_h.at[pl.ds(row0 * GROUP_LIMIT, _S_PER * GROUP_LIMIT)])
    pltpu.sync_copy(ogy_v, gy_h.at[pl.ds(row0 * GROUP_LIMIT, _S_PER * GROUP_LIMIT)])
    pltpu.sync_copy(ogz_v, gz_h.at[pl.ds(row0 * GROUP_LIMIT, _S_PER * GROUP_LIMIT)])
    pltpu.sync_copy(ocnt_v, cnt_h.at[pl.ds(row0 * _LANES, _S_PER * _LANES)])


def _ballquery_sc(xs, ys, zs, bxs, bys, bzs, sqv, bcx, bcy, bcz, sqc):
    f32 = jnp.float32
    mesh = plsc.VectorSubcoreMesh(core_axis_name="c", subcore_axis_name="s")
    sg = jax.ShapeDtypeStruct((SUBSAMPLE_SIZE * GROUP_LIMIT,), f32)
    kern = pl.kernel(
        _bq_body,
        out_type=[sg, sg, sg,
                  jax.ShapeDtypeStruct((SUBSAMPLE_SIZE * _LANES,), jnp.int32)],
        mesh=mesh,
        compiler_params=pltpu.CompilerParams(needs_layout_passes=False),
        scratch_types=[
            pltpu.VMEM((N_POINTS,), f32),
            pltpu.VMEM((N_POINTS,), f32),
            pltpu.VMEM((N_POINTS,), f32),
            pltpu.VMEM((N_POINTS,), f32),
            pltpu.VMEM((N_POINTS,), f32),
            pltpu.VMEM((N_POINTS,), f32),
            pltpu.VMEM((N_POINTS,), f32),
            pltpu.VMEM((_S_PER * _LANES,), f32),
            pltpu.VMEM((_S_PER * _LANES,), f32),
            pltpu.VMEM((_S_PER * _LANES,), f32),
            pltpu.VMEM((_S_PER * _LANES,), f32),
            pltpu.VMEM((_CAND,), jnp.int32),
            pltpu.VMEM((_CAND,), f32),
            pltpu.VMEM((_CAND,), f32),
            pltpu.VMEM((_CAND,), f32),
            pltpu.VMEM((_CAND,), f32),
            pltpu.VMEM((_S_PER * GROUP_LIMIT,), f32),
            pltpu.VMEM((_S_PER * GROUP_LIMIT,), f32),
            pltpu.VMEM((_S_PER * GROUP_LIMIT,), f32),
            pltpu.VMEM((_S_PER * _LANES,), jnp.int32),
        ],
    )
    return kern(xs, ys, zs, bxs, bys, bzs, sqv, bcx, bcy, bcz, sqc)


def _mlp_kernel(feat_ref, mask_ref, w1_ref, b1_ref, w2_ref, b2_ref, w3_ref, b3_ref, out_ref):
    feat = feat_ref[...]
    h = jnp.maximum(feat @ w1_ref[...] + b1_ref[...], 0.0)
    h = jnp.maximum(h @ w2_ref[...] + b2_ref[...], 0.0)
    h = h @ w3_ref[...] + b3_ref[...]
    h = jnp.where(mask_ref[...] > 0, h, -1e9)
    BSK = h.shape[0]
    out_ref[...] = jnp.max(h.reshape(BSK // GROUP_LIMIT, GROUP_LIMIT, 128), axis=1)


def _mlp_pallas(feat8, mask_i, W1p, b1, W2, b2, W3, b3):
    S = SUBSAMPLE_SIZE
    BS = 128
    return pl.pallas_call(
        _mlp_kernel,
        grid=(S // BS,),
        in_specs=[
            pl.BlockSpec((BS * GROUP_LIMIT, 8), lambda i: (i, 0)),
            pl.BlockSpec((BS * GROUP_LIMIT, 1), lambda i: (i, 0)),
            pl.BlockSpec((8, 64), lambda i: (0, 0)),
            pl.BlockSpec((64,), lambda i: (0,)),
            pl.BlockSpec((64, 64), lambda i: (0, 0)),
            pl.BlockSpec((64,), lambda i: (0,)),
            pl.BlockSpec((64, 128), lambda i: (0, 0)),
            pl.BlockSpec((128,), lambda i: (0,)),
        ],
        out_specs=pl.BlockSpec((BS, 128), lambda i: (i, 0)),
        out_shape=jax.ShapeDtypeStruct((S, 128), jnp.float32),
    )(feat8, mask_i, W1p, b1, W2, b2, W3, b3)


def kernel(vertices, W1, b1, W2, b2, W3, b3):
    S = SUBSAMPLE_SIZE
    K = GROUP_LIMIT
    cx, cy, cz = _fps_pallas(vertices)

    xs = vertices[:, 0]
    ys = vertices[:, 1]
    zs = vertices[:, 2]
    def bf16_round(a):
        return lax.reduce_precision(a, exponent_bits=8, mantissa_bits=7)

    bxs = bf16_round(xs)
    bys = bf16_round(ys)
    bzs = bf16_round(zs)
    sqv = jnp.sum(vertices * vertices, axis=-1)
    sqc = cx * cx + cy * cy + cz * cz

    def splat(a):
        return jnp.broadcast_to(a[:, None], (S, _LANES)).reshape(-1)

    bcx = splat(bf16_round(cx))
    bcy = splat(bf16_round(cy))
    bcz = splat(bf16_round(cz))
    gx, gy, gz, cnt16 = _ballquery_sc(
        xs, ys, zs, bxs, bys, bzs, sqv, bcx, bcy, bcz, splat(sqc))
    gx = gx.reshape(S, K)
    gy = gy.reshape(S, K)
    gz = gz.reshape(S, K)
    cnt = cnt16.reshape(S, _LANES)[:, 0]

    feat8 = jnp.stack(
        [gx, gy, gz, gx - cx[:, None], gy - cy[:, None], gz - cz[:, None],
         jnp.zeros((S, K), jnp.float32), jnp.zeros((S, K), jnp.float32)],
        axis=-1,
    ).reshape(S * K, 8)
    kidx = jnp.arange(K, dtype=jnp.int32)[None, :]
    mask_i = (kidx < cnt[:, None]).astype(jnp.int32).reshape(S * K, 1)
    W1p = jnp.concatenate([W1, jnp.zeros((2, 64), jnp.float32)], axis=0)
    return _mlp_pallas(feat8, mask_i, W1p, b1, W2, b2, W3, b3)


# speedup vs baseline: 1.7540x; 1.7540x over previous
import functools

import numpy as np
import jax
import jax.numpy as jnp
from jax import lax
from jax.experimental import pallas as pl
from jax.experimental.pallas import tpu as pltpu
from jax.experimental.pallas import tpu_sc as plsc

SUBSAMPLE_SIZE = 1024
GROUP_LIMIT = 64
N_POINTS = 16384

_LANES = 16
_NW = 32
_S_PER = SUBSAMPLE_SIZE // _NW
_CAND = 512
_T04 = float(np.float32(0.040000003))


def _fps_body(x_ref, y_ref, z_ref, cx_ref, cy_ref, cz_ref):
    x = x_ref[...]
    y = y_ref[...]
    z = z_ref[...]
    ri = lax.broadcasted_iota(jnp.int32, (128, 128), 0)
    ci = lax.broadcasted_iota(jnp.int32, (128, 128), 1)
    flat = ri * 128 + ci

    def extract(a, idx):
        return jnp.sum(jnp.where(flat == idx, a, 0.0))

    lx0 = extract(x, 0)
    ly0 = extract(y, 0)
    lz0 = extract(z, 0)
    cx_ref[0] = lx0
    cy_ref[0] = ly0
    cz_ref[0] = lz0
    dmin0 = jnp.full((128, 128), jnp.inf, dtype=jnp.float32)

    def body(i, carry):
        dmin, lx, ly, lz = carry
        dx = x - lx
        dy = y - ly
        dz = z - lz
        d = dx * dx + dy * dy + dz * dz
        dmin = jnp.minimum(dmin, d)
        m = jnp.max(dmin)
        idx = jnp.min(jnp.where(dmin == m, flat, jnp.int32(1 << 30)))
        nlx = extract(x, idx)
        nly = extract(y, idx)
        nlz = extract(z, idx)
        cx_ref[i] = nlx
        cy_ref[i] = nly
        cz_ref[i] = nlz
        return dmin, nlx, nly, nlz

    lax.fori_loop(1, SUBSAMPLE_SIZE, body, (dmin0, lx0, ly0, lz0))


def _fps_pallas(vertices):
    xp = vertices[:, 0].reshape(128, 128)
    yp = vertices[:, 1].reshape(128, 128)
    zp = vertices[:, 2].reshape(128, 128)
    s = jax.ShapeDtypeStruct((SUBSAMPLE_SIZE,), jnp.float32)
    return pl.pallas_call(
        _fps_body,
        out_specs=[pl.BlockSpec(memory_space=pltpu.SMEM)] * 3,
        out_shape=[s, s, s],
    )(xp, yp, zp)


def _bq_body(xs_h, ys_h, zs_h, bxs_h, bys_h, bzs_h, sqv_h,
             bcx_h, bcy_h, bcz_h, sqc_h,
             gx_h, gy_h, gz_h, cnt_h,
             xs_v, ys_v, zs_v, bxs_v, bys_v, bzs_v, sqv_v,
             bcxv, bcyv, bczv, sqcv,
             cid_v, ca2_v, cgx_v, cgy_v, cgz_v,
             ogx_v, ogy_v, ogz_v, ocnt_v):
    i32 = jnp.int32
    f32 = jnp.float32
    wid = lax.axis_index("s") * 2 + lax.axis_index("c")
    row0 = wid * _S_PER
    pltpu.sync_copy(xs_h, xs_v)
    pltpu.sync_copy(ys_h, ys_v)
    pltpu.sync_copy(zs_h, zs_v)
    pltpu.sync_copy(bxs_h, bxs_v)
    pltpu.sync_copy(bys_h, bys_v)
    pltpu.sync_copy(bzs_h, bzs_v)
    pltpu.sync_copy(sqv_h, sqv_v)
    pltpu.sync_copy(bcx_h.at[pl.ds(row0 * _LANES, _S_PER * _LANES)], bcxv)
    pltpu.sync_copy(bcy_h.at[pl.ds(row0 * _LANES, _S_PER * _LANES)], bcyv)
    pltpu.sync_copy(bcz_h.at[pl.ds(row0 * _LANES, _S_PER * _LANES)], bczv)
    pltpu.sync_copy(sqc_h.at[pl.ds(row0 * _LANES, _S_PER * _LANES)], sqcv)

    lane = lax.broadcasted_iota(i32, (_LANES,), 0)
    zero16i = jnp.zeros((_LANES,), i32)
    one16i = jnp.ones((_LANES,), i32)
    t04 = jnp.full((_LANES,), _T04, f32)
    inf16 = jnp.full((_LANES,), jnp.inf, f32)

    def per_centroid(j, _):
        cxs = bcxv[pl.ds(j * _LANES, _LANES)]
        cys = bcyv[pl.ds(j * _LANES, _LANES)]
        czs = bczv[pl.ds(j * _LANES, _LANES)]
        sqcs = sqcv[pl.ds(j * _LANES, _LANES)]

        @plsc.parallel_loop(0, N_POINTS // _LANES, unroll=4, carry=zero16i)
        def cnt_v(i, cnt_v):
            base = i * _LANES
            xv = bxs_v[pl.ds(base, _LANES)]
            yv = bys_v[pl.ds(base, _LANES)]
            zv = bzs_v[pl.ds(base, _LANES)]
            sv = sqv_v[pl.ds(base, _LANES)]
            dot = xv * cxs + yv * cys + zv * czs
            a2 = jnp.abs(sqcs - 2.0 * dot + sv)
            m = a2 <= t04
            cum = plsc.cumsum(jnp.where(m, one16i, zero16i))
            slot = cnt_v + cum - 1
            m2 = jnp.logical_and(m, slot < _CAND)
            plsc.store_scatter(cid_v, [slot], lane + base, mask=m2)
            return cnt_v + plsc.all_reduce_population_count(m2)
        c = jnp.max(cnt_v)
        cbase = j * GROUP_LIMIT
        zf = jnp.zeros((_LANES,), f32)
        for k in range(GROUP_LIMIT // _LANES):
            ogx_v[pl.ds(cbase + k * _LANES, _LANES)] = zf
            ogy_v[pl.ds(cbase + k * _LANES, _LANES)] = zf
            ogz_v[pl.ds(cbase + k * _LANES, _LANES)] = zf
        nch = (c + _LANES - 1) // _LANES

        def gath(i, _):
            valid = (i * _LANES + lane) < cnt_v
            idxv = cid_v[pl.ds(i * _LANES, _LANES)]
            idxc = jnp.where(valid, idxv, zero16i)
            bxv = plsc.load_gather(bxs_v, [idxc])
            byv = plsc.load_gather(bys_v, [idxc])
            bzv = plsc.load_gather(bzs_v, [idxc])
            sv = plsc.load_gather(sqv_v, [idxc])
            dot = bxv * cxs + byv * cys + bzv * czs
            a2 = jnp.abs(sqcs - 2.0 * dot + sv)
            ca2_v[pl.ds(i * _LANES, _LANES)] = jnp.where(valid, a2, inf16)
            cgx_v[pl.ds(i * _LANES, _LANES)] = plsc.load_gather(xs_v, [idxc])
            cgy_v[pl.ds(i * _LANES, _LANES)] = plsc.load_gather(ys_v, [idxc])
            cgz_v[pl.ds(i * _LANES, _LANES)] = plsc.load_gather(zs_v, [idxc])
            return 0

        lax.fori_loop(0, nch, gath, 0)

        def bsearch():
            def bs(_, lohi):
                lo, hi = lohi
                mid = 0.5 * (lo + hi)

                def cntit(i, acc):
                    a2v = ca2_v[pl.ds(i * _LANES, _LANES)]
                    return acc + plsc.all_reduce_population_count(a2v <= mid)

                cv = lax.fori_loop(0, nch, cntit, zero16i)
                ge = cv >= GROUP_LIMIT
                return jnp.where(ge, lo, mid), jnp.where(ge, mid, hi)

            lo0 = jnp.zeros((_LANES,), f32)
            return lax.fori_loop(0, 40, bs, (lo0, t04))[1]

        t = lax.cond(c > GROUP_LIMIT, bsearch, lambda: t04)

        def sel(i, taken):
            valid = (i * _LANES + lane) < cnt_v
            a2v = ca2_v[pl.ds(i * _LANES, _LANES)]
            m = jnp.logical_and(a2v <= t, valid)
            cum = plsc.cumsum(jnp.where(m, one16i, zero16i))
            slot = taken + cum - 1
            m2 = jnp.logical_and(m, slot < GROUP_LIMIT)
            gxv = cgx_v[pl.ds(i * _LANES, _LANES)]
            gyv = cgy_v[pl.ds(i * _LANES, _LANES)]
            gzv = cgz_v[pl.ds(i * _LANES, _LANES)]
            plsc.store_scatter(ogx_v, [cbase + slot], gxv, mask=m2)
            plsc.store_scatter(ogy_v, [cbase + slot], gyv, mask=m2)
            plsc.store_scatter(ogz_v, [cbase + slot], gzv, mask=m2)
            return taken + plsc.all_reduce_population_count(m2)

        lax.fori_loop(0, nch, sel, zero16i)
        ocnt_v[pl.ds(j * _LANES, _LANES)] = jnp.minimum(cnt_v, GROUP_LIMIT)
        return 0

    lax.fori_loop(0, _S_PER, per_centroid, 0)
    pltpu.sync_copy(ogx_v, gx_h.at[pl.ds(row0 * GROUP_LIMIT, _S_PER * GROUP_LIMIT)])
    pltpu.sync_copy(ogy_v, gy_h.at[pl.ds(row0 * GROUP_LIMIT, _S_PER * GROUP_LIMIT)])
    pltpu.sync_copy(ogz_v, gz_h.at[pl.ds(row0 * GROUP_LIMIT, _S_PER * GROUP_LIMIT)])
    pltpu.sync_copy(ocnt_v, cnt_h.at[pl.ds(row0 * _LANES, _S_PER * _LANES)])


def _ballquery_sc(xs, ys, zs, bxs, bys, bzs, sqv, bcx, bcy, bcz, sqc):
    f32 = jnp.float32
    mesh = plsc.VectorSubcoreMesh(core_axis_name="c", subcore_axis_name="s")
    sg = jax.ShapeDtypeStruct((SUBSAMPLE_SIZE * GROUP_LIMIT,), f32)
    kern = pl.kernel(
        _bq_body,
        out_type=[sg, sg, sg,
                  jax.ShapeDtypeStruct((SUBSAMPLE_SIZE * _LANES,), jnp.int32)],
        mesh=mesh,
        compiler_params=pltpu.CompilerParams(needs_layout_passes=False),
        scratch_types=[
            pltpu.VMEM((N_POINTS,), f32),
            pltpu.VMEM((N_POINTS,), f32),
            pltpu.VMEM((N_POINTS,), f32),
            pltpu.VMEM((N_POINTS,), f32),
            pltpu.VMEM((N_POINTS,), f32),
            pltpu.VMEM((N_POINTS,), f32),
            pltpu.VMEM((N_POINTS,), f32),
            pltpu.VMEM((_S_PER * _LANES,), f32),
            pltpu.VMEM((_S_PER * _LANES,), f32),
            pltpu.VMEM((_S_PER * _LANES,), f32),
            pltpu.VMEM((_S_PER * _LANES,), f32),
            pltpu.VMEM((_CAND,), jnp.int32),
            pltpu.VMEM((_CAND,), f32),
            pltpu.VMEM((_CAND,), f32),
            pltpu.VMEM((_CAND,), f32),
            pltpu.VMEM((_CAND,), f32),
            pltpu.VMEM((_S_PER * GROUP_LIMIT,), f32),
            pltpu.VMEM((_S_PER * GROUP_LIMIT,), f32),
            pltpu.VMEM((_S_PER * GROUP_LIMIT,), f32),
            pltpu.VMEM((_S_PER * _LANES,), jnp.int32),
        ],
    )
    return kern(xs, ys, zs, bxs, bys, bzs, sqv, bcx, bcy, bcz, sqc)


def _mlp_kernel(feat_ref, mask_ref, w1_ref, b1_ref, w2_ref, b2_ref, w3_ref, b3_ref, out_ref):
    feat = feat_ref[...]
    h = jnp.maximum(feat @ w1_ref[...] + b1_ref[...], 0.0)
    h = jnp.maximum(h @ w2_ref[...] + b2_ref[...], 0.0)
    h = h @ w3_ref[...] + b3_ref[...]
    h = jnp.where(mask_ref[...] > 0, h, -1e9)
    BSK = h.shape[0]
    out_ref[...] = jnp.max(h.reshape(BSK // GROUP_LIMIT, GROUP_LIMIT, 128), axis=1)


def _mlp_pallas(feat8, mask_i, W1p, b1, W2, b2, W3, b3):
    S = SUBSAMPLE_SIZE
    BS = 128
    return pl.pallas_call(
        _mlp_kernel,
        grid=(S // BS,),
        in_specs=[
            pl.BlockSpec((BS * GROUP_LIMIT, 8), lambda i: (i, 0)),
            pl.BlockSpec((BS * GROUP_LIMIT, 1), lambda i: (i, 0)),
            pl.BlockSpec((8, 64), lambda i: (0, 0)),
            pl.BlockSpec((64,), lambda i: (0,)),
            pl.BlockSpec((64, 64), lambda i: (0, 0)),
            pl.BlockSpec((64,), lambda i: (0,)),
            pl.BlockSpec((64, 128), lambda i: (0, 0)),
            pl.BlockSpec((128,), lambda i: (0,)),
        ],
        out_specs=pl.BlockSpec((BS, 128), lambda i: (i, 0)),
        out_shape=jax.ShapeDtypeStruct((S, 128), jnp.float32),
    )(feat8, mask_i, W1p, b1, W2, b2, W3, b3)


def kernel(vertices, W1, b1, W2, b2, W3, b3):
    S = SUBSAMPLE_SIZE
    K = GROUP_LIMIT
    cx, cy, cz = _fps_pallas(vertices)

    xs = vertices[:, 0]
    ys = vertices[:, 1]
    zs = vertices[:, 2]
    def bf16_round(a):
        return lax.reduce_precision(a, exponent_bits=8, mantissa_bits=7)

    bxs = bf16_round(xs)
    bys = bf16_round(ys)
    bzs = bf16_round(zs)
    sqv = jnp.sum(vertices * vertices, axis=-1)
    sqc = cx * cx + cy * cy + cz * cz

    def splat(a):
        return jnp.broadcast_to(a[:, None], (S, _LANES)).reshape(-1)

    bcx = splat(bf16_round(cx))
    bcy = splat(bf16_round(cy))
    bcz = splat(bf16_round(cz))
    gx, gy, gz, cnt16 = _ballquery_sc(
        xs, ys, zs, bxs, bys, bzs, sqv, bcx, bcy, bcz, splat(sqc))
    gx = gx.reshape(S, K)
    gy = gy.reshape(S, K)
    gz = gz.reshape(S, K)
    cnt = cnt16.reshape(S, _LANES)[:, 0]

    feat8 = jnp.stack(
        [gx, gy, gz, gx - cx[:, None], gy - cy[:, None], gz - cz[:, None],
         jnp.zeros((S, K), jnp.float32), jnp.zeros((S, K), jnp.float32)],
        axis=-1,
    ).reshape(S * K, 8)
    kidx = jnp.arange(K, dtype=jnp.int32)[None, :]
    mask_i = (kidx < cnt[:, None]).astype(jnp.int32).reshape(S * K, 1)
    W1p = jnp.concatenate([W1, jnp.zeros((2, 64), jnp.float32)], axis=0)
    return _mlp_pallas(feat8, mask_i, W1p, b1, W2, b2, W3, b3)
